# baseline (device time: 45678 ns/iter reference)
import jax
import jax.numpy as jnp
from jax import lax
from jax.experimental import pallas as pl
from jax.experimental.pallas import tpu as pltpu

N_Z = 4


def kernel(partial, resid, gamma):
    _, m, d = partial.shape
    mh = m // 2
    mc = mh // N_Z
    gamma2 = gamma.reshape(1, d)

    def body(p_ref, r_ref, g_ref, out_ref,
             rs_send, rs_recv, ag_send, ag_recv, xg_buf,
             rs_send_sems, rs_recv_sems, ag_send_sems, ag_recv_sems,
             x_send_sems, x_recv_sems):
        my_x = lax.axis_index("x")
        my_y = lax.axis_index("y")
        my_z = lax.axis_index("z")
        xn = 1 - my_x
        my_base = my_x * mh
        nb_base = xn * mh

        barrier_sem = pltpu.get_barrier_semaphore()
        for k in range(1, N_Z):
            pl.semaphore_signal(
                barrier_sem, inc=1,
                device_id=(my_x, my_y, (my_z + k) % N_Z),
                device_id_type=pl.DeviceIdType.MESH,
            )
        pl.semaphore_signal(
            barrier_sem, inc=1,
            device_id=(xn, my_y, my_z),
            device_id_type=pl.DeviceIdType.MESH,
        )
        pl.semaphore_wait(barrier_sem, N_Z)

        def my_chunk(c):
            return p_ref[0, pl.ds(my_base + c * mc, mc), :]

        rs_rdmas = []
        for k in range(1, N_Z):
            r = (my_z - k) % N_Z
            rs_send[k - 1] = my_chunk(r).astype(jnp.bfloat16)
            rdma = pltpu.make_async_remote_copy(
                src_ref=rs_send.at[k - 1],
                dst_ref=rs_recv.at[k - 1],
                send_sem=rs_send_sems.at[k - 1],
                recv_sem=rs_recv_sems.at[k - 1],
                device_id=(my_x, my_y, r),
                device_id_type=pl.DeviceIdType.MESH,
            )
            rdma.start()
            rs_rdmas.append(rdma)

        for rdma in rs_rdmas:
            rdma.wait_recv()
        total = my_chunk(my_z)
        for j in range(N_Z - 1):
            total = total + rs_recv[j].astype(jnp.float32)

        y = total + r_ref[pl.ds(my_base + my_z * mc, mc), :]
        rms = jnp.sqrt(jnp.mean(y * y, axis=-1, keepdims=True) + 1e-6)
        norm = (y / rms) * g_ref[...]
        ag_send[...] = norm.astype(jnp.bfloat16)
        out_ref[pl.ds(my_base + my_z * mc, mc), :] = norm

        ag_rdmas = []
        for k in range(1, N_Z):
            r = (my_z - k) % N_Z
            rdma = pltpu.make_async_remote_copy(
                src_ref=ag_send,
                dst_ref=ag_recv.at[k - 1],
                send_sem=ag_send_sems.at[k - 1],
                recv_sem=ag_recv_sems.at[k - 1],
                device_id=(my_x, my_y, r),
                device_id_type=pl.DeviceIdType.MESH,
            )
            rdma.start()
            ag_rdmas.append(rdma)

        def x_push(slot, src):
            rdma = pltpu.make_async_remote_copy(
                src_ref=src,
                dst_ref=xg_buf.at[slot],
                send_sem=x_send_sems.at[slot],
                recv_sem=x_recv_sems.at[slot],
                device_id=(xn, my_y, my_z),
                device_id_type=pl.DeviceIdType.MESH,
            )
            rdma.start()
            return rdma

        x_rdmas = [x_push(0, ag_send)]

        for j in range(N_Z - 1):
            ag_rdmas[j].wait_recv()
            x_rdmas.append(x_push(j + 1, ag_recv.at[j]))
            c = (my_z + j + 1) % N_Z
            out_ref[pl.ds(my_base + c * mc, mc), :] = (
                ag_recv[j].astype(jnp.float32))

        x_rdmas[0].wait_recv()
        out_ref[pl.ds(nb_base + my_z * mc, mc), :] = xg_buf[0].astype(jnp.float32)
        for j in range(N_Z - 1):
            x_rdmas[j + 1].wait_recv()
            c = (my_z + j + 1) % N_Z
            out_ref[pl.ds(nb_base + c * mc, mc), :] = (
                xg_buf[j + 1].astype(jnp.float32))

        for rdma in rs_rdmas + ag_rdmas + x_rdmas:
            rdma.wait_send()

    return pl.pallas_call(
        body,
        out_shape=jax.ShapeDtypeStruct((m, d), jnp.float32),
        in_specs=[
            pl.BlockSpec(memory_space=pltpu.VMEM),
            pl.BlockSpec(memory_space=pltpu.VMEM),
            pl.BlockSpec(memory_space=pltpu.VMEM),
        ],
        out_specs=pl.BlockSpec(memory_space=pltpu.VMEM),
        scratch_shapes=[
            pltpu.VMEM((N_Z - 1, mc, d), jnp.bfloat16),
            pltpu.VMEM((N_Z - 1, mc, d), jnp.bfloat16),
            pltpu.VMEM((mc, d), jnp.bfloat16),
            pltpu.VMEM((N_Z - 1, mc, d), jnp.bfloat16),
            pltpu.VMEM((N_Z, mc, d), jnp.bfloat16),
            pltpu.SemaphoreType.DMA((N_Z - 1,)),
            pltpu.SemaphoreType.DMA((N_Z - 1,)),
            pltpu.SemaphoreType.DMA((N_Z - 1,)),
            pltpu.SemaphoreType.DMA((N_Z - 1,)),
            pltpu.SemaphoreType.DMA((N_Z,)),
            pltpu.SemaphoreType.DMA((N_Z,)),
        ],
        compiler_params=pltpu.CompilerParams(collective_id=0),
    )(partial, resid, gamma2)
